# baseline (speedup 1.0000x reference)
import jax
import jax.numpy as jnp
from jax import lax
from jax.experimental import pallas as pl
from jax.experimental.pallas import tpu as pltpu
from jax.experimental.pallas import tpu_sc as plsc

N = 10000
E = 320000
D_IN = 128
D_HID = 256
D_OUT = 128
D = 128

NC = 2
NS = 16
B = 128

E_LOOP = E + N
R_SC = 88
R_TOT = NC * NS * R_SC
E_PAD = R_TOT * B
N_TRASH = 128
N_TAB = N + N_TRASH

CH = 80
NCH = N // CH
CPT = (NCH + NS - 1) // NS
IGRP = 8
NGRP = R_SC // IGRP


def _mesh():
    return plsc.VectorSubcoreMesh(
        core_axis_name="c", subcore_axis_name="s", num_cores=NC, num_subcores=NS
    )


def _edge_loop(table, src_hbm, dst_hbm, acc, sidx, didx, buf0, buf1,
               sem0, sem1, base_rows, ngrp):
    bufs = (buf0, buf1)
    sems = (sem0, sem1)

    @pl.loop(0, ngrp)
    def _(g):
        off_i = pl.multiple_of(base_rows + g * IGRP, 8)
        pltpu.sync_copy(src_hbm.at[pl.ds(off_i, IGRP)], sidx)
        pltpu.sync_copy(dst_hbm.at[pl.ds(off_i, IGRP)], didx)
        descs = [None] * IGRP
        descs[0] = pltpu.async_copy(table.at[sidx.at[0]], bufs[0], sems[0])
        for j in range(IGRP):
            if j + 1 < IGRP:
                descs[j + 1] = pltpu.async_copy(
                    table.at[sidx.at[j + 1]], bufs[(j + 1) % 2],
                    sems[(j + 1) % 2])
            descs[j].wait()
            pltpu.sync_copy(bufs[j % 2], acc.at[didx.at[j]], add=True)


def _chunked(s, copy_fn):
    @pl.loop(0, CPT)
    def _(k):
        cid = s + k * NS

        @pl.when(cid < NCH)
        def _():
            copy_fn(pl.multiple_of(cid * CH, 8))


def _deg_body(dst_hbm, zeros_hbm, ones_hbm, out0_hbm, out1_hbm,
              deg_acc, didx, ones_v, stage):
    c = lax.axis_index("c")
    s = lax.axis_index("s")
    w = c * NS + s
    pltpu.sync_copy(dst_hbm.at[pl.ds(pl.multiple_of(w * R_SC, 8), R_SC)], didx)
    pltpu.sync_copy(ones_hbm, ones_v)

    pltpu.sync_copy(zeros_hbm, stage)

    def zinit(off):
        pltpu.sync_copy(stage, deg_acc.at[pl.ds(off, CH)])
    _chunked(s, zinit)

    plsc.subcore_barrier()

    @pl.loop(0, R_SC)
    def _(j):
        pltpu.sync_copy(ones_v, deg_acc.at[didx.at[j]], add=True)

    plsc.subcore_barrier()

    def wb(out_hbm):
        def cp(off):
            pltpu.sync_copy(deg_acc.at[pl.ds(off, CH)], stage)
            pltpu.sync_copy(stage, out_hbm.at[pl.ds(off, CH)])
        _chunked(s, cp)

    @pl.when(c == 0)
    def _():
        wb(out0_hbm)

    @pl.when(c == 1)
    def _():
        wb(out1_hbm)


def _deg_kernel(dstp, zeros, ones):
    kfn = pl.kernel(
        _deg_body,
        out_type=[
            jax.ShapeDtypeStruct((N,), jnp.float32),
            jax.ShapeDtypeStruct((N,), jnp.float32),
        ],
        mesh=_mesh(),
        scratch_types=[
            pltpu.VMEM_SHARED((N_TAB,), jnp.float32),
            pltpu.VMEM((R_SC, B), jnp.int32),
            pltpu.VMEM((B,), jnp.float32),
            pltpu.VMEM((CH,), jnp.float32),
        ],
    )
    return kfn(dstp, zeros, ones)


def _scatter_body(y, zeros2_hbm, src_hbm, dst_hbm, out_hbm, acc, sidx, didx,
                  buf0, buf1, stage, sem0, sem1):
    c = lax.axis_index("c")
    s = lax.axis_index("s")

    pltpu.sync_copy(zeros2_hbm, stage)

    def zinit(off):
        pltpu.sync_copy(stage, acc.at[pl.ds(off, CH)])
    _chunked(s, zinit)

    plsc.subcore_barrier()

    _edge_loop(y, src_hbm, dst_hbm, acc, sidx, didx, buf0, buf1,
               sem0, sem1, (c * NS + s) * R_SC, NGRP)

    plsc.subcore_barrier()

    def wb(off):
        pltpu.sync_copy(acc.at[pl.ds(off, CH)], stage)
        pltpu.sync_copy(stage, out_hbm.at[c, pl.ds(off, CH)])
    _chunked(s, wb)


def _scatter_kernel(y, zeros2, srcp, dstp):
    kfn = pl.kernel(
        _scatter_body,
        out_type=jax.ShapeDtypeStruct((NC, N, D), jnp.float32),
        mesh=_mesh(),
        scratch_types=[
            pltpu.VMEM_SHARED((N_TAB, D), jnp.float32),
            pltpu.VMEM((IGRP, B), jnp.int32),
            pltpu.VMEM((IGRP, B), jnp.int32),
            pltpu.VMEM((B, D), jnp.float32),
            pltpu.VMEM((B, D), jnp.float32),
            pltpu.VMEM((CH, D), jnp.float32),
            pltpu.SemaphoreType.DMA,
            pltpu.SemaphoreType.DMA,
        ],
    )
    return kfn(y, zeros2, srcp, dstp)


def _dis(deg0_ref, deg1_ref):
    deg = deg0_ref[...] + deg1_ref[...]
    return lax.rsqrt(deg)[:, None]


def _tca_body(x_ref, deg0_ref, deg1_ref, xt_ref):
    xt_ref[...] = x_ref[...] * _dis(deg0_ref, deg1_ref)


def _tca(x, deg0, deg1):
    return pl.pallas_call(
        _tca_body,
        out_shape=jax.ShapeDtypeStruct((N, D_IN), jnp.float32),
    )(x, deg0, deg1)


def _tcb_body(agg_ref, deg0_ref, deg1_ref, w1_ref, b1_ref, w2_ref, y_ref):
    dis = _dis(deg0_ref, deg1_ref)
    agg = (agg_ref[0] + agg_ref[1]) * dis
    h = jnp.maximum(
        jnp.dot(agg, w1_ref[...], preferred_element_type=jnp.float32)
        + b1_ref[...][None, :], 0.0)
    y_ref[...] = jnp.dot(h, w2_ref[...],
                         preferred_element_type=jnp.float32) * dis


def _tcb(agg1, deg0, deg1, W1, b1, W2):
    return pl.pallas_call(
        _tcb_body,
        out_shape=jax.ShapeDtypeStruct((N, D_OUT), jnp.float32),
    )(agg1, deg0, deg1, W1, b1, W2)


def _tcc_body(agg_ref, deg0_ref, deg1_ref, b2_ref, out_ref):
    agg = agg_ref[0] + agg_ref[1]
    out_ref[...] = jnp.maximum(
        agg * _dis(deg0_ref, deg1_ref) + b2_ref[...][None, :], 0.0)


def _tcc(agg2, deg0, deg1, b2):
    return pl.pallas_call(
        _tcc_body,
        out_shape=jax.ShapeDtypeStruct((N, D_OUT), jnp.float32),
    )(agg2, deg0, deg1, b2)


def kernel(x, edge_index, W1, b1, W2, b2):
    ei = edge_index.astype(jnp.int32)
    loop = jnp.arange(N, dtype=jnp.int32)
    pad = E_PAD - E_LOOP
    srcp = jnp.concatenate(
        [ei[0], loop, jnp.zeros((pad,), jnp.int32)]).reshape(R_TOT, B)
    trash = N + (jnp.arange(pad, dtype=jnp.int32) % N_TRASH)
    dstp = jnp.concatenate([ei[1], loop, trash]).reshape(R_TOT, B)
    zeros = jnp.zeros((CH,), jnp.float32)
    zeros2 = jnp.zeros((CH, D), jnp.float32)
    ones = jnp.ones((B,), jnp.float32)

    deg0, deg1 = _deg_kernel(dstp, zeros, ones)
    xt = _tca(x, deg0, deg1)
    agg1 = _scatter_kernel(xt, zeros2, srcp, dstp)
    y2 = _tcb(agg1, deg0, deg1, W1, b1, W2)
    agg2 = _scatter_kernel(y2, zeros2, srcp, dstp)
    return _tcc(agg2, deg0, deg1, b2)


# speedup vs baseline: 8.1995x; 8.1995x over previous
import jax
import jax.numpy as jnp
from jax import lax
from jax.experimental import pallas as pl
from jax.experimental.pallas import tpu as pltpu
from jax.experimental.pallas import tpu_sc as plsc

N = 10000
E = 320000
D_IN = 128
D_HID = 256
D_OUT = 128
D = 128

NC = 2
NS = 16
B = 128

E_LOOP = E + N
R_SC = 88
R_TOT = NC * NS * R_SC
E_PAD = R_TOT * B
N_TRASH = 128
N_TAB = N + N_TRASH

CH = 80
NCH = N // CH
CPT = (NCH + NS - 1) // NS
IGRP = 8
NGRP = R_SC // IGRP


def _mesh():
    return plsc.VectorSubcoreMesh(
        core_axis_name="c", subcore_axis_name="s", num_cores=NC, num_subcores=NS
    )


def _edge_loop(table, src_hbm, dst_hbm, acc, sidx, didx, buf0, buf1,
               sem0, sem1, base_rows, ngrp):
    bufs = (buf0, buf1)
    sems = (sem0, sem1)

    @pl.loop(0, ngrp)
    def _(g):
        off_i = pl.multiple_of(base_rows + g * IGRP, 8)
        pltpu.sync_copy(src_hbm.at[pl.ds(off_i, IGRP)], sidx)
        pltpu.sync_copy(dst_hbm.at[pl.ds(off_i, IGRP)], didx)
        descs = [None] * IGRP
        descs[0] = pltpu.async_copy(table.at[sidx.at[0]], bufs[0], sems[0])
        for j in range(IGRP):
            if j + 1 < IGRP:
                descs[j + 1] = pltpu.async_copy(
                    table.at[sidx.at[j + 1]], bufs[(j + 1) % 2],
                    sems[(j + 1) % 2])
            descs[j].wait()
            pltpu.sync_copy(bufs[j % 2], acc.at[didx.at[j]], add=True)


def _chunked(s, copy_fn):
    @pl.loop(0, CPT)
    def _(k):
        cid = s + k * NS

        @pl.when(cid < NCH)
        def _():
            copy_fn(pl.multiple_of(cid * CH, 8))


def _deg_body(dst_hbm, zeros_hbm, ones_hbm, out0_hbm, out1_hbm,
              deg_acc, didx, ones_v, stage):
    c = lax.axis_index("c")
    s = lax.axis_index("s")
    w = c * NS + s
    pltpu.sync_copy(dst_hbm.at[pl.ds(pl.multiple_of(w * R_SC, 8), R_SC)], didx)
    pltpu.sync_copy(ones_hbm, ones_v)

    pltpu.sync_copy(zeros_hbm, stage)

    def zinit(off):
        pltpu.sync_copy(stage, deg_acc.at[pl.ds(off, CH)])
    _chunked(s, zinit)

    plsc.subcore_barrier()

    @pl.loop(0, R_SC)
    def _(j):
        pltpu.sync_copy(ones_v, deg_acc.at[didx.at[j]], add=True)

    plsc.subcore_barrier()

    def wb(out_hbm):
        def cp(off):
            pltpu.sync_copy(deg_acc.at[pl.ds(off, CH)], stage)
            pltpu.sync_copy(stage, out_hbm.at[pl.ds(off, CH)])
        _chunked(s, cp)

    @pl.when(c == 0)
    def _():
        wb(out0_hbm)

    @pl.when(c == 1)
    def _():
        wb(out1_hbm)


def _deg_kernel(dstp, zeros, ones):
    kfn = pl.kernel(
        _deg_body,
        out_type=[
            jax.ShapeDtypeStruct((N,), jnp.float32),
            jax.ShapeDtypeStruct((N,), jnp.float32),
        ],
        mesh=_mesh(),
        scratch_types=[
            pltpu.VMEM_SHARED((N_TAB,), jnp.float32),
            pltpu.VMEM((R_SC, B), jnp.int32),
            pltpu.VMEM((B,), jnp.float32),
            pltpu.VMEM((CH,), jnp.float32),
        ],
    )
    return kfn(dstp, zeros, ones)


def _scatter_body(y, zeros2_hbm, src_hbm, dst_hbm, out_hbm, acc, sidx, didx,
                  buf0, buf1, stage, sem0, sem1):
    c = lax.axis_index("c")
    s = lax.axis_index("s")

    pltpu.sync_copy(zeros2_hbm, stage)

    def zinit(off):
        pltpu.sync_copy(stage, acc.at[pl.ds(off, CH)])
    _chunked(s, zinit)

    plsc.subcore_barrier()

    _edge_loop(y, src_hbm, dst_hbm, acc, sidx, didx, buf0, buf1,
               sem0, sem1, (c * NS + s) * R_SC, NGRP)

    plsc.subcore_barrier()

    def wb(off):
        pltpu.sync_copy(acc.at[pl.ds(off, CH)], stage)
        pltpu.sync_copy(stage, out_hbm.at[c, pl.ds(off, CH)])
    _chunked(s, wb)


def _scatter_kernel(y, zeros2, srcp, dstp):
    kfn = pl.kernel(
        _scatter_body,
        out_type=jax.ShapeDtypeStruct((NC, N, D), jnp.float32),
        mesh=_mesh(),
        scratch_types=[
            pltpu.VMEM_SHARED((N_TAB, D), jnp.float32),
            pltpu.VMEM((IGRP, B), jnp.int32),
            pltpu.VMEM((IGRP, B), jnp.int32),
            pltpu.VMEM((B, D), jnp.float32),
            pltpu.VMEM((B, D), jnp.float32),
            pltpu.VMEM((CH, D), jnp.float32),
            pltpu.SemaphoreType.DMA,
            pltpu.SemaphoreType.DMA,
        ],
    )
    return kfn(y, zeros2, srcp, dstp)


def _dis(deg0_ref, deg1_ref):
    deg = deg0_ref[...] + deg1_ref[...]
    return lax.rsqrt(deg)[:, None]


def _tca_body(x_ref, deg0_ref, deg1_ref, xt_ref):
    xt_ref[...] = x_ref[...] * _dis(deg0_ref, deg1_ref)


def _tca(x, deg0, deg1):
    return pl.pallas_call(
        _tca_body,
        out_shape=jax.ShapeDtypeStruct((N, D_IN), jnp.float32),
    )(x, deg0, deg1)


def _tcb_body(agg_ref, deg0_ref, deg1_ref, w1_ref, b1_ref, w2_ref, y_ref):
    dis = _dis(deg0_ref, deg1_ref)
    agg = (agg_ref[0] + agg_ref[1]) * dis
    h = jnp.maximum(
        jnp.dot(agg, w1_ref[...], preferred_element_type=jnp.float32)
        + b1_ref[...][None, :], 0.0)
    y_ref[...] = jnp.dot(h, w2_ref[...],
                         preferred_element_type=jnp.float32) * dis


def _tcb(agg1, deg0, deg1, W1, b1, W2):
    return pl.pallas_call(
        _tcb_body,
        out_shape=jax.ShapeDtypeStruct((N, D_OUT), jnp.float32),
    )(agg1, deg0, deg1, W1, b1, W2)


def _tcc_body(agg_ref, deg0_ref, deg1_ref, b2_ref, out_ref):
    agg = agg_ref[0] + agg_ref[1]
    out_ref[...] = jnp.maximum(
        agg * _dis(deg0_ref, deg1_ref) + b2_ref[...][None, :], 0.0)


def _tcc(agg2, deg0, deg1, b2):
    return pl.pallas_call(
        _tcc_body,
        out_shape=jax.ShapeDtypeStruct((N, D_OUT), jnp.float32),
    )(agg2, deg0, deg1, b2)


def kernel(x, edge_index, W1, b1, W2, b2):
    ei = edge_index.astype(jnp.int32)
    loop = jnp.arange(N, dtype=jnp.int32)
    pad = E_PAD - E_LOOP
    psrc = jnp.arange(pad, dtype=jnp.int32) * 37 % N
    trash = N + (jnp.arange(pad, dtype=jnp.int32) % N_TRASH)
    srcp = (jnp.concatenate([ei[0], loop, psrc]).reshape(R_SC, NC * NS, B)
            .swapaxes(0, 1).reshape(R_TOT, B))
    dstp = (jnp.concatenate([ei[1], loop, trash]).reshape(R_SC, NC * NS, B)
            .swapaxes(0, 1).reshape(R_TOT, B))
    zeros = jnp.zeros((CH,), jnp.float32)
    zeros2 = jnp.zeros((CH, D), jnp.float32)
    ones = jnp.ones((B,), jnp.float32)

    deg0, deg1 = _deg_kernel(dstp, zeros, ones)
    xt = _tca(x, deg0, deg1)
    agg1 = _scatter_kernel(xt, zeros2, srcp, dstp)
    y2 = _tcb(agg1, deg0, deg1, W1, b1, W2)
    agg2 = _scatter_kernel(y2, zeros2, srcp, dstp)
    return _tcc(agg2, deg0, deg1, b2)


# speedup vs baseline: 8.4750x; 1.0336x over previous
import jax
import jax.numpy as jnp
from jax import lax
from jax.experimental import pallas as pl
from jax.experimental.pallas import tpu as pltpu
from jax.experimental.pallas import tpu_sc as plsc

N = 10000
E = 320000
D_IN = 128
D_HID = 256
D_OUT = 128
D = 128

NC = 2
NS = 16
B = 128

E_LOOP = E + N
R_SC = 88
R_TOT = NC * NS * R_SC
E_PAD = R_TOT * B
N_TRASH = 128
N_TAB = N + N_TRASH

CH = 80
NCH = N // CH
CPT = (NCH + NS - 1) // NS
IGRP = 8
NGRP = R_SC // IGRP


def _mesh():
    return plsc.VectorSubcoreMesh(
        core_axis_name="c", subcore_axis_name="s", num_cores=NC, num_subcores=NS
    )


def _edge_loop(table, idx_hbm, acc, sdidx, buf0, buf1,
               sem0, sem1, base_grp, ngrp):
    bufs = (buf0, buf1)
    sems = (sem0, sem1)

    @pl.loop(0, ngrp)
    def _(g):
        off_i = pl.multiple_of((base_grp + g) * 2 * IGRP, 8)
        pltpu.sync_copy(idx_hbm.at[pl.ds(off_i, 2 * IGRP)], sdidx)
        descs = [None] * IGRP
        descs[0] = pltpu.async_copy(table.at[sdidx.at[0]], bufs[0], sems[0])
        for j in range(IGRP):
            if j + 1 < IGRP:
                descs[j + 1] = pltpu.async_copy(
                    table.at[sdidx.at[j + 1]], bufs[(j + 1) % 2],
                    sems[(j + 1) % 2])
            descs[j].wait()
            pltpu.sync_copy(bufs[j % 2], acc.at[sdidx.at[IGRP + j]], add=True)


def _chunked(s, copy_fn):
    @pl.loop(0, CPT)
    def _(k):
        cid = s + k * NS

        @pl.when(cid < NCH)
        def _():
            copy_fn(pl.multiple_of(cid * CH, 8))


def _deg_body(dst_hbm, zeros_hbm, ones_hbm, out0_hbm, out1_hbm,
              deg_acc, didx, ones_v, stage):
    c = lax.axis_index("c")
    s = lax.axis_index("s")
    w = c * NS + s
    pltpu.sync_copy(dst_hbm.at[pl.ds(pl.multiple_of(w * R_SC, 8), R_SC)], didx)
    pltpu.sync_copy(ones_hbm, ones_v)

    pltpu.sync_copy(zeros_hbm, stage)

    def zinit(off):
        pltpu.sync_copy(stage, deg_acc.at[pl.ds(off, CH)])
    _chunked(s, zinit)

    plsc.subcore_barrier()

    @pl.loop(0, R_SC)
    def _(j):
        pltpu.sync_copy(ones_v, deg_acc.at[didx.at[j]], add=True)

    plsc.subcore_barrier()

    def wb(out_hbm):
        def cp(off):
            pltpu.sync_copy(deg_acc.at[pl.ds(off, CH)], stage)
            pltpu.sync_copy(stage, out_hbm.at[pl.ds(off, CH)])
        _chunked(s, cp)

    @pl.when(c == 0)
    def _():
        wb(out0_hbm)

    @pl.when(c == 1)
    def _():
        wb(out1_hbm)


def _deg_kernel(dstp, zeros, ones):
    kfn = pl.kernel(
        _deg_body,
        out_type=[
            jax.ShapeDtypeStruct((N,), jnp.float32),
            jax.ShapeDtypeStruct((N,), jnp.float32),
        ],
        mesh=_mesh(),
        scratch_types=[
            pltpu.VMEM_SHARED((N_TAB,), jnp.float32),
            pltpu.VMEM((R_SC, B), jnp.int32),
            pltpu.VMEM((B,), jnp.float32),
            pltpu.VMEM((CH,), jnp.float32),
        ],
    )
    return kfn(dstp, zeros, ones)


def _scatter_body(y, zeros2_hbm, idx_hbm, out_hbm, acc, sdidx,
                  buf0, buf1, stage, sem0, sem1):
    c = lax.axis_index("c")
    s = lax.axis_index("s")

    pltpu.sync_copy(zeros2_hbm, stage)

    def zinit(off):
        pltpu.sync_copy(stage, acc.at[pl.ds(off, CH)])
    _chunked(s, zinit)

    plsc.subcore_barrier()

    _edge_loop(y, idx_hbm, acc, sdidx, buf0, buf1,
               sem0, sem1, (c * NS + s) * NGRP, NGRP)

    plsc.subcore_barrier()

    def wb(off):
        pltpu.sync_copy(acc.at[pl.ds(off, CH)], stage)
        pltpu.sync_copy(stage, out_hbm.at[c, pl.ds(off, CH)])
    _chunked(s, wb)


def _scatter_kernel(y, zeros2, idxc):
    kfn = pl.kernel(
        _scatter_body,
        out_type=jax.ShapeDtypeStruct((NC, N, D), jnp.float32),
        mesh=_mesh(),
        scratch_types=[
            pltpu.VMEM_SHARED((N_TAB, D), jnp.float32),
            pltpu.VMEM((2 * IGRP, B), jnp.int32),
            pltpu.VMEM((B, D), jnp.float32),
            pltpu.VMEM((B, D), jnp.float32),
            pltpu.VMEM((CH, D), jnp.float32),
            pltpu.SemaphoreType.DMA,
            pltpu.SemaphoreType.DMA,
        ],
    )
    return kfn(y, zeros2, idxc)


def _dis(deg0_ref, deg1_ref):
    deg = deg0_ref[...] + deg1_ref[...]
    return lax.rsqrt(deg)[:, None]


def _tca_body(x_ref, deg0_ref, deg1_ref, xt_ref):
    xt_ref[...] = x_ref[...] * _dis(deg0_ref, deg1_ref)


def _tca(x, deg0, deg1):
    return pl.pallas_call(
        _tca_body,
        out_shape=jax.ShapeDtypeStruct((N, D_IN), jnp.float32),
    )(x, deg0, deg1)


def _tcb_body(agg_ref, deg0_ref, deg1_ref, w1_ref, b1_ref, w2_ref, y_ref):
    dis = _dis(deg0_ref, deg1_ref)
    agg = (agg_ref[0] + agg_ref[1]) * dis
    h = jnp.maximum(
        jnp.dot(agg, w1_ref[...], preferred_element_type=jnp.float32)
        + b1_ref[...][None, :], 0.0)
    y_ref[...] = jnp.dot(h, w2_ref[...],
                         preferred_element_type=jnp.float32) * dis


def _tcb(agg1, deg0, deg1, W1, b1, W2):
    return pl.pallas_call(
        _tcb_body,
        out_shape=jax.ShapeDtypeStruct((N, D_OUT), jnp.float32),
    )(agg1, deg0, deg1, W1, b1, W2)


def _tcc_body(agg_ref, deg0_ref, deg1_ref, b2_ref, out_ref):
    agg = agg_ref[0] + agg_ref[1]
    out_ref[...] = jnp.maximum(
        agg * _dis(deg0_ref, deg1_ref) + b2_ref[...][None, :], 0.0)


def _tcc(agg2, deg0, deg1, b2):
    return pl.pallas_call(
        _tcc_body,
        out_shape=jax.ShapeDtypeStruct((N, D_OUT), jnp.float32),
    )(agg2, deg0, deg1, b2)


def kernel(x, edge_index, W1, b1, W2, b2):
    ei = edge_index.astype(jnp.int32)
    loop = jnp.arange(N, dtype=jnp.int32)
    pad = E_PAD - E_LOOP
    psrc = jnp.arange(pad, dtype=jnp.int32) * 37 % N
    trash = N + (jnp.arange(pad, dtype=jnp.int32) % N_TRASH)
    srcp = (jnp.concatenate([ei[0], loop, psrc]).reshape(R_SC, NC * NS, B)
            .swapaxes(0, 1).reshape(R_TOT, B))
    dstp = (jnp.concatenate([ei[1], loop, trash]).reshape(R_SC, NC * NS, B)
            .swapaxes(0, 1).reshape(R_TOT, B))
    idxc = jnp.concatenate(
        [srcp.reshape(-1, IGRP, B), dstp.reshape(-1, IGRP, B)], axis=1
    ).reshape(2 * R_TOT, B)
    zeros = jnp.zeros((CH,), jnp.float32)
    zeros2 = jnp.zeros((CH, D), jnp.float32)
    ones = jnp.ones((B,), jnp.float32)

    deg0, deg1 = _deg_kernel(dstp, zeros, ones)
    xt = _tca(x, deg0, deg1)
    agg1 = _scatter_kernel(xt, zeros2, idxc)
    y2 = _tcb(agg1, deg0, deg1, W1, b1, W2)
    agg2 = _scatter_kernel(y2, zeros2, idxc)
    return _tcc(agg2, deg0, deg1, b2)


# speedup vs baseline: 8.7854x; 1.0366x over previous
import jax
import jax.numpy as jnp
from jax import lax
from jax.experimental import pallas as pl
from jax.experimental.pallas import tpu as pltpu
from jax.experimental.pallas import tpu_sc as plsc

N = 10000
E = 320000
D_IN = 128
D_HID = 256
D_OUT = 128
D = 128

NC = 2
NS = 16
B = 128

E_LOOP = E + N
R_SC = 88
R_TOT = NC * NS * R_SC
E_PAD = R_TOT * B
N_TRASH = 128
N_TAB = N + N_TRASH

CH = 80
NCH = N // CH
CPT = (NCH + NS - 1) // NS
IGRP = 8
NGRP = R_SC // IGRP


def _mesh():
    return plsc.VectorSubcoreMesh(
        core_axis_name="c", subcore_axis_name="s", num_cores=NC, num_subcores=NS
    )


def _edge_loop(table, idx_hbm, acc, sdidx0, sdidx1, buf0, buf1,
               sem0, sem1, isem, base_grp, ngrp):
    bufs = (buf0, buf1)
    sems = (sem0, sem1)

    def group_body(g, sdidx, sdidx_next, prefetch):
        ipdesc = None
        if prefetch:
            off_n = pl.multiple_of((base_grp + g + 1) * 2 * IGRP, 8)
            ipdesc = pltpu.async_copy(
                idx_hbm.at[pl.ds(off_n, 2 * IGRP)], sdidx_next, isem)
        descs = [None] * IGRP
        descs[0] = pltpu.async_copy(table.at[sdidx.at[0]], bufs[0], sems[0])
        for j in range(IGRP):
            if j + 1 < IGRP:
                descs[j + 1] = pltpu.async_copy(
                    table.at[sdidx.at[j + 1]], bufs[(j + 1) % 2],
                    sems[(j + 1) % 2])
            descs[j].wait()
            pltpu.sync_copy(bufs[j % 2], acc.at[sdidx.at[IGRP + j]], add=True)
        if prefetch:
            ipdesc.wait()

    pltpu.sync_copy(
        idx_hbm.at[pl.ds(pl.multiple_of(base_grp * 2 * IGRP, 8), 2 * IGRP)],
        sdidx0)

    @pl.loop(0, (ngrp - 1) // 2)
    def _(t):
        g = 2 * t
        group_body(g, sdidx0, sdidx1, True)
        group_body(g + 1, sdidx1, sdidx0, True)

    group_body(ngrp - 1, sdidx0, sdidx1, False)


def _chunked(s, copy_fn):
    @pl.loop(0, CPT)
    def _(k):
        cid = s + k * NS

        @pl.when(cid < NCH)
        def _():
            copy_fn(pl.multiple_of(cid * CH, 8))


def _deg_body(dst_hbm, zeros_hbm, ones_hbm, out0_hbm, out1_hbm,
              deg_acc, didx, ones_v, stage):
    c = lax.axis_index("c")
    s = lax.axis_index("s")
    w = c * NS + s
    pltpu.sync_copy(dst_hbm.at[pl.ds(pl.multiple_of(w * R_SC, 8), R_SC)], didx)
    pltpu.sync_copy(ones_hbm, ones_v)

    pltpu.sync_copy(zeros_hbm, stage)

    def zinit(off):
        pltpu.sync_copy(stage, deg_acc.at[pl.ds(off, CH)])
    _chunked(s, zinit)

    plsc.subcore_barrier()

    @pl.loop(0, R_SC)
    def _(j):
        pltpu.sync_copy(ones_v, deg_acc.at[didx.at[j]], add=True)

    plsc.subcore_barrier()

    def wb(out_hbm):
        def cp(off):
            pltpu.sync_copy(deg_acc.at[pl.ds(off, CH)], stage)
            pltpu.sync_copy(stage, out_hbm.at[pl.ds(off, CH)])
        _chunked(s, cp)

    @pl.when(c == 0)
    def _():
        wb(out0_hbm)

    @pl.when(c == 1)
    def _():
        wb(out1_hbm)


def _deg_kernel(dstp, zeros, ones):
    kfn = pl.kernel(
        _deg_body,
        out_type=[
            jax.ShapeDtypeStruct((N,), jnp.float32),
            jax.ShapeDtypeStruct((N,), jnp.float32),
        ],
        mesh=_mesh(),
        scratch_types=[
            pltpu.VMEM_SHARED((N_TAB,), jnp.float32),
            pltpu.VMEM((R_SC, B), jnp.int32),
            pltpu.VMEM((B,), jnp.float32),
            pltpu.VMEM((CH,), jnp.float32),
        ],
    )
    return kfn(dstp, zeros, ones)


def _scatter_body(y, zeros2_hbm, idx_hbm, out_hbm, acc, sdidx0, sdidx1,
                  buf0, buf1, stage, sem0, sem1, isem):
    c = lax.axis_index("c")
    s = lax.axis_index("s")

    pltpu.sync_copy(zeros2_hbm, stage)

    def zinit(off):
        pltpu.sync_copy(stage, acc.at[pl.ds(off, CH)])
    _chunked(s, zinit)

    plsc.subcore_barrier()

    _edge_loop(y, idx_hbm, acc, sdidx0, sdidx1, buf0, buf1,
               sem0, sem1, isem, (c * NS + s) * NGRP, NGRP)

    plsc.subcore_barrier()

    def wb(off):
        pltpu.sync_copy(acc.at[pl.ds(off, CH)], stage)
        pltpu.sync_copy(stage, out_hbm.at[c, pl.ds(off, CH)])
    _chunked(s, wb)


def _scatter_kernel(y, zeros2, idxc):
    kfn = pl.kernel(
        _scatter_body,
        out_type=jax.ShapeDtypeStruct((NC, N, D), jnp.float32),
        mesh=_mesh(),
        scratch_types=[
            pltpu.VMEM_SHARED((N_TAB, D), jnp.float32),
            pltpu.VMEM((2 * IGRP, B), jnp.int32),
            pltpu.VMEM((2 * IGRP, B), jnp.int32),
            pltpu.VMEM((B, D), jnp.float32),
            pltpu.VMEM((B, D), jnp.float32),
            pltpu.VMEM((CH, D), jnp.float32),
            pltpu.SemaphoreType.DMA,
            pltpu.SemaphoreType.DMA,
            pltpu.SemaphoreType.DMA,
        ],
    )
    return kfn(y, zeros2, idxc)


def _dis(deg0_ref, deg1_ref):
    deg = deg0_ref[...] + deg1_ref[...]
    return lax.rsqrt(deg)[:, None]


def _tca_body(x_ref, deg0_ref, deg1_ref, xt_ref):
    xt_ref[...] = x_ref[...] * _dis(deg0_ref, deg1_ref)


def _tca(x, deg0, deg1):
    return pl.pallas_call(
        _tca_body,
        out_shape=jax.ShapeDtypeStruct((N, D_IN), jnp.float32),
    )(x, deg0, deg1)


def _tcb_body(agg_ref, deg0_ref, deg1_ref, w1_ref, b1_ref, w2_ref, y_ref):
    dis = _dis(deg0_ref, deg1_ref)
    agg = (agg_ref[0] + agg_ref[1]) * dis
    h = jnp.maximum(
        jnp.dot(agg, w1_ref[...], preferred_element_type=jnp.float32)
        + b1_ref[...][None, :], 0.0)
    y_ref[...] = jnp.dot(h, w2_ref[...],
                         preferred_element_type=jnp.float32) * dis


def _tcb(agg1, deg0, deg1, W1, b1, W2):
    return pl.pallas_call(
        _tcb_body,
        out_shape=jax.ShapeDtypeStruct((N, D_OUT), jnp.float32),
    )(agg1, deg0, deg1, W1, b1, W2)


def _tcc_body(agg_ref, deg0_ref, deg1_ref, b2_ref, out_ref):
    agg = agg_ref[0] + agg_ref[1]
    out_ref[...] = jnp.maximum(
        agg * _dis(deg0_ref, deg1_ref) + b2_ref[...][None, :], 0.0)


def _tcc(agg2, deg0, deg1, b2):
    return pl.pallas_call(
        _tcc_body,
        out_shape=jax.ShapeDtypeStruct((N, D_OUT), jnp.float32),
    )(agg2, deg0, deg1, b2)


def kernel(x, edge_index, W1, b1, W2, b2):
    ei = edge_index.astype(jnp.int32)
    loop = jnp.arange(N, dtype=jnp.int32)
    pad = E_PAD - E_LOOP
    psrc = jnp.arange(pad, dtype=jnp.int32) * 37 % N
    trash = N + (jnp.arange(pad, dtype=jnp.int32) % N_TRASH)
    srcp = (jnp.concatenate([ei[0], loop, psrc]).reshape(R_SC, NC * NS, B)
            .swapaxes(0, 1).reshape(R_TOT, B))
    dstp = (jnp.concatenate([ei[1], loop, trash]).reshape(R_SC, NC * NS, B)
            .swapaxes(0, 1).reshape(R_TOT, B))
    idxc = jnp.concatenate(
        [srcp.reshape(-1, IGRP, B), dstp.reshape(-1, IGRP, B)], axis=1
    ).reshape(2 * R_TOT, B)
    zeros = jnp.zeros((CH,), jnp.float32)
    zeros2 = jnp.zeros((CH, D), jnp.float32)
    ones = jnp.ones((B,), jnp.float32)

    deg0, deg1 = _deg_kernel(dstp, zeros, ones)
    xt = _tca(x, deg0, deg1)
    agg1 = _scatter_kernel(xt, zeros2, idxc)
    y2 = _tcb(agg1, deg0, deg1, W1, b1, W2)
    agg2 = _scatter_kernel(y2, zeros2, idxc)
    return _tcc(agg2, deg0, deg1, b2)
